# Initial kernel scaffold; baseline (speedup 1.0000x reference)
import jax
import jax.numpy as jnp
from jax.experimental import pallas as pl


def kernel(bias, index):
    raise NotImplementedError("write your pallas kernel here")


# speedup vs baseline: 2.5339x; 2.5339x over previous
import functools

import jax
import jax.numpy as jnp
from jax import lax
from jax.experimental import pallas as pl
from jax.experimental.pallas import tpu as pltpu
from jax.experimental.pallas import tpu_sc as plsc

H = 16
U = 729
N = 38416
L = 16
NW = 32
STEP = 1200
CHUNK = 1216
NVREG = CHUNK // L


_mesh = plsc.VectorSubcoreMesh(core_axis_name="c", subcore_axis_name="s")


@functools.partial(
    pl.kernel,
    mesh=_mesh,
    compiler_params=pltpu.CompilerParams(
        needs_layout_passes=False, use_tc_tiling_on_sc=False
    ),
    out_type=jax.ShapeDtypeStruct((H * N,), jnp.float32),
    scratch_types=[
        pltpu.VMEM((H * U,), jnp.float32),
        pltpu.VMEM((CHUNK,), jnp.int32),
        pltpu.VMEM((H, CHUNK), jnp.float32),
    ],
)
def _gather_bias(bias_hbm, idx_hbm, out_hbm, bias_v, idx_v, out_v):
    cid = lax.axis_index("c")
    sid = lax.axis_index("s")
    wid = sid * 2 + cid
    base = wid * STEP

    pltpu.sync_copy(bias_hbm, bias_v)
    pltpu.sync_copy(idx_hbm.at[pl.ds(base, CHUNK)], idx_v)

    def step(v, carry):
        iv = idx_v[pl.ds(v * L, L)]
        for h in range(H):
            out_v[h, pl.ds(v * L, L)] = plsc.load_gather(bias_v, [iv + h * U])
        return carry

    lax.fori_loop(0, NVREG, step, 0)

    for h in range(H):
        pltpu.sync_copy(out_v.at[h], out_hbm.at[pl.ds(h * N + base, CHUNK)])


def kernel(bias, index):
    out = _gather_bias(bias.reshape(H * U), index)
    return out.reshape(H, 196, 196)


# speedup vs baseline: 2.5677x; 1.0134x over previous
import functools

import jax
import jax.numpy as jnp
from jax import lax
from jax.experimental import pallas as pl
from jax.experimental.pallas import tpu as pltpu
from jax.experimental.pallas import tpu_sc as plsc

H = 16
U = 729
N = 38416
L = 16
NW = 32
STEP = 1200
CHUNK = 1216
NVREG = CHUNK // L


_mesh = plsc.VectorSubcoreMesh(core_axis_name="c", subcore_axis_name="s")


@functools.partial(
    pl.kernel,
    mesh=_mesh,
    compiler_params=pltpu.CompilerParams(
        needs_layout_passes=False, use_tc_tiling_on_sc=False
    ),
    out_type=jax.ShapeDtypeStruct((H * N,), jnp.float32),
    scratch_types=[
        pltpu.VMEM((H * U,), jnp.float32),
        pltpu.VMEM((CHUNK,), jnp.int32),
        pltpu.VMEM((H, CHUNK), jnp.float32),
        pltpu.SemaphoreType.DMA,
        pltpu.SemaphoreType.DMA,
        pltpu.SemaphoreType.DMA,
    ],
)
def _gather_bias(bias_hbm, idx_hbm, out_hbm, bias_v, idx_v, out_v,
                 sem_b, sem_i, sem_o):
    cid = lax.axis_index("c")
    sid = lax.axis_index("s")
    wid = sid * 2 + cid
    base = wid * STEP

    cp_b = pltpu.async_copy(bias_hbm, bias_v, sem_b)
    cp_i = pltpu.async_copy(idx_hbm.at[pl.ds(base, CHUNK)], idx_v, sem_i)
    cp_i.wait()
    cp_b.wait()

    UNROLL = 4
    out_cps = []
    for h in range(H):
        def step(vo, carry, h=h):
            for k in range(UNROLL):
                v = vo * UNROLL + k
                iv = idx_v[pl.ds(v * L, L)]
                out_v[h, pl.ds(v * L, L)] = plsc.load_gather(
                    bias_v, [iv + h * U])
            return carry

        lax.fori_loop(0, NVREG // UNROLL, step, 0)
        out_cps.append(
            pltpu.async_copy(out_v.at[h], out_hbm.at[pl.ds(h * N + base, CHUNK)],
                             sem_o))
    for cp in out_cps:
        cp.wait()


def kernel(bias, index):
    out = _gather_bias(bias.reshape(H * U), index)
    return out.reshape(H, 196, 196)


# speedup vs baseline: 2.5806x; 1.0050x over previous
import functools

import jax
import jax.numpy as jnp
from jax import lax
from jax.experimental import pallas as pl
from jax.experimental.pallas import tpu as pltpu
from jax.experimental.pallas import tpu_sc as plsc

H = 16
U = 729
N = 38416
L = 16
NW = 32
STEP = 1200
CHUNK = 1216
NVREG = CHUNK // L


_mesh = plsc.VectorSubcoreMesh(core_axis_name="c", subcore_axis_name="s")


@functools.partial(
    pl.kernel,
    mesh=_mesh,
    compiler_params=pltpu.CompilerParams(
        needs_layout_passes=False, use_tc_tiling_on_sc=False
    ),
    out_type=jax.ShapeDtypeStruct((H * N,), jnp.float32),
    scratch_types=[
        pltpu.VMEM((H * U,), jnp.float32),
        pltpu.VMEM((CHUNK,), jnp.int32),
        pltpu.VMEM((H, CHUNK), jnp.float32),
        pltpu.SemaphoreType.DMA,
        pltpu.SemaphoreType.DMA,
        pltpu.SemaphoreType.DMA,
    ],
)
def _gather_bias(bias_hbm, idx_hbm, out_hbm, bias_v, idx_v, out_v,
                 sem_b, sem_i, sem_o):
    cid = lax.axis_index("c")
    sid = lax.axis_index("s")
    wid = sid * 2 + cid
    base = wid * STEP

    cp_b = pltpu.async_copy(bias_hbm, bias_v, sem_b)
    cp_i = pltpu.async_copy(idx_hbm.at[pl.ds(base, CHUNK)], idx_v, sem_i)
    cp_i.wait()
    cp_b.wait()

    HALF = CHUNK // 2
    HV = HALF // L
    UNROLL = 2
    out_cps = []
    for half in range(2):
        off = half * HALF

        def step(vo, carry, off=off):
            for k in range(UNROLL):
                s = off + (vo * UNROLL + k) * L
                iv = idx_v[pl.ds(s, L)]
                for h in range(H):
                    out_v[h, pl.ds(s, L)] = plsc.load_gather(
                        bias_v, [iv + h * U])
            return carry

        lax.fori_loop(0, HV // UNROLL, step, 0)
        for h in range(H):
            out_cps.append(
                pltpu.async_copy(
                    out_v.at[h, pl.ds(off, HALF)],
                    out_hbm.at[pl.ds(h * N + base + off, HALF)],
                    sem_o))
    for cp in out_cps:
        cp.wait()


def kernel(bias, index):
    out = _gather_bias(bias.reshape(H * U), index)
    return out.reshape(H, 196, 196)
